# baseline (speedup 1.0000x reference)
import functools

import jax
import jax.numpy as jnp
from jax import lax
from jax.experimental import pallas as pl
from jax.experimental.pallas import tpu as pltpu, tpu_sc as plsc

_VOCAB = 10000
_HALF = 64
_NC, _NS = 2, 16
_NW = _NC * _NS
_PARTS = ((0, 128), (128, 72))
_NP = len(_PARTS)


@functools.partial(jax.jit, static_argnames=("B", "L"))
def _sc_embed(pos, W_now, W_next, B, L):
    b_per_w = B // _NW
    total_q = b_per_w * _NP
    mesh = plsc.VectorSubcoreMesh(core_axis_name="c", subcore_axis_name="s")

    @functools.partial(
        pl.kernel,
        out_type=jax.ShapeDtypeStruct((B * L, 2, _HALF), jnp.float32),
        mesh=mesh,
        scratch_types=[
            pltpu.VMEM((2, 2, L), jnp.int32),
            pltpu.VMEM((_PARTS[0][1], _HALF), jnp.float32),
            pltpu.VMEM((_PARTS[1][1], _HALF), jnp.float32),
            pltpu.VMEM((_PARTS[0][1], _HALF), jnp.float32),
            pltpu.VMEM((_PARTS[1][1], _HALF), jnp.float32),
            pltpu.VMEM_SHARED((_VOCAB, _HALF), jnp.float32),
            pltpu.VMEM_SHARED((_VOCAB, _HALF), jnp.float32),
            pltpu.SemaphoreType.DMA,
            pltpu.SemaphoreType.DMA,
            pltpu.SemaphoreType.DMA,
            pltpu.SemaphoreType.DMA,
            pltpu.SemaphoreType.DMA,
            pltpu.SemaphoreType.DMA,
            pltpu.SemaphoreType.DMA,
            pltpu.SemaphoreType.DMA,
        ],
        compiler_params=pltpu.CompilerParams(use_tc_tiling_on_sc=False),
    )
    def k(pos_hbm, wn_hbm, wx_hbm, out_hbm, idx_v, bn0, bn1, bx0, bx1,
          wn_sp, wx_sp, isem0, isem1, gsem0, gsem1, on0, on1, ox0, ox1):
        bns, bxs = (bn0, bn1), (bx0, bx1)
        isems, gsems = (isem0, isem1), (gsem0, gsem1)
        osems_n, osems_x = (on0, on1), (ox0, ox1)
        wid = lax.axis_index("s") * _NC + lax.axis_index("c")
        b0 = wid * b_per_w

        sid = lax.axis_index("s")
        stripe = _VOCAB // _NS
        pltpu.sync_copy(wn_hbm.at[pl.ds(sid * stripe, stripe)],
                        wn_sp.at[pl.ds(sid * stripe, stripe)])
        pltpu.sync_copy(wx_hbm.at[pl.ds(sid * stripe, stripe)],
                        wx_sp.at[pl.ds(sid * stripe, stripe)])
        plsc.subcore_barrier()

        def idx_load(b, t):
            return pltpu.make_async_copy(pos_hbm.at[b0 + b], idx_v.at[t],
                                         isems[t])

        def gathers_start(b_t, part, s):
            off, ln = _PARTS[part]
            pltpu.async_copy(wn_sp.at[idx_v.at[b_t, 0, pl.ds(off, ln)]],
                             bns[s], gsems[s])
            pltpu.async_copy(wx_sp.at[idx_v.at[b_t, 1, pl.ds(off, ln)]],
                             bxs[s], gsems[s])

        def gathers_wait(part, s):
            _, ln = _PARTS[part]
            pltpu.make_async_copy(wn_hbm.at[pl.ds(0, ln)],
                                  bns[s], gsems[s]).wait()
            pltpu.make_async_copy(wx_hbm.at[pl.ds(0, ln)],
                                  bxs[s], gsems[s]).wait()

        def store_now(b, part, s):
            off, ln = _PARTS[part]
            return pltpu.make_async_copy(
                bns[s],
                out_hbm.at[pl.ds((b0 + b) * L + off, ln), 0], osems_n[s])

        def store_next(b, part, s):
            off, ln = _PARTS[part]
            return pltpu.make_async_copy(
                bxs[s],
                out_hbm.at[pl.ds((b0 + b) * L + off, ln), 1], osems_x[s])

        idx_load(0, 0).start()
        idx_load(1, 1).start()

        @pl.loop(0, total_q, step=2 * _NP)
        def _body(g):
            for u in range(2 * _NP):
                q = g + u
                part = u % _NP
                s = u % 2
                b = q // _NP
                b_t = (u // _NP) % 2

                if part == 0:
                    idx_load(0, b_t).wait()

                @pl.when(q >= 2)
                def _():
                    store_now(0, part, s).wait()
                    store_next(0, part, s).wait()

                gathers_start(b_t, part, s)

                part_m1 = (u - 1) % _NP
                s_m1 = 1 - s
                b_t_m1 = ((u - 1) % (2 * _NP)) // _NP

                @pl.when(q >= 1)
                def _():
                    b_m1 = jnp.maximum(q - 1, 0) // _NP
                    gathers_wait(part_m1, s_m1)
                    store_now(b_m1, part_m1, s_m1).start()
                    store_next(b_m1, part_m1, s_m1).start()

                if part_m1 == _NP - 1:
                    @pl.when((q >= 1) & (q - 1 + 2 * _NP < total_q))
                    def _():
                        idx_load(jnp.maximum(q - 1, 0) // _NP + 2, b_t_m1).start()

        q_last = total_q - 1
        gathers_wait(q_last % _NP, q_last % 2)
        store_now(b_per_w - 1, q_last % _NP, q_last % 2).start()
        store_next(b_per_w - 1, q_last % _NP, q_last % 2).start()
        for q in (total_q - 2, total_q - 1):
            store_now(0, q % _NP, q % 2).wait()
            store_next(0, q % _NP, q % 2).wait()

    return k(pos, W_now, W_next)


def kernel(pos, W_now, W_next):
    B, _, L = pos.shape
    out = _sc_embed(pos.astype(jnp.int32), W_now, W_next, B, L)
    return out.reshape(B, L, 2 * _HALF)


# speedup vs baseline: 1.1908x; 1.1908x over previous
import functools

import jax
import jax.numpy as jnp
from jax import lax
from jax.experimental import pallas as pl
from jax.experimental.pallas import tpu as pltpu, tpu_sc as plsc

_VOCAB = 10000
_HALF = 64
_NC, _NS = 2, 16
_NW = _NC * _NS
_PARTS = ((0, 96), (96, 104))
_PMAX = 104
_NP = len(_PARTS)


@functools.partial(jax.jit, static_argnames=("B", "L"))
def _sc_embed(pos, W_now, W_next, B, L):
    b_per_w = B // _NW
    total_q = b_per_w * _NP
    mesh = plsc.VectorSubcoreMesh(core_axis_name="c", subcore_axis_name="s")

    @functools.partial(
        pl.kernel,
        out_type=jax.ShapeDtypeStruct((B * L, 2, _HALF), jnp.float32),
        mesh=mesh,
        scratch_types=[
            pltpu.VMEM((2, 2, L), jnp.int32),
            pltpu.VMEM((2, _PMAX, _HALF), jnp.float32),
            pltpu.VMEM((2, _PMAX, _HALF), jnp.float32),
            pltpu.VMEM_SHARED((_VOCAB, _HALF), jnp.float32),
            pltpu.VMEM_SHARED((_VOCAB, _HALF), jnp.float32),
            pltpu.SemaphoreType.DMA,
            pltpu.SemaphoreType.DMA,
            pltpu.SemaphoreType.DMA,
            pltpu.SemaphoreType.DMA,
            pltpu.SemaphoreType.DMA,
            pltpu.SemaphoreType.DMA,
            pltpu.SemaphoreType.DMA,
            pltpu.SemaphoreType.DMA,
        ],
        compiler_params=pltpu.CompilerParams(use_tc_tiling_on_sc=False),
    )
    def k(pos_hbm, wn_hbm, wx_hbm, out_hbm, idx_v, bn_v, bx_v, wn_sp, wx_sp,
          isem0, isem1, gsem0, gsem1, on0, on1, ox0, ox1):
        isems, gsems = (isem0, isem1), (gsem0, gsem1)
        osems_n, osems_x = (on0, on1), (ox0, ox1)
        wid = lax.axis_index("s") * _NC + lax.axis_index("c")
        b0 = wid * b_per_w

        sid = lax.axis_index("s")
        stripe = _VOCAB // _NS
        pltpu.sync_copy(wn_hbm.at[pl.ds(sid * stripe, stripe)],
                        wn_sp.at[pl.ds(sid * stripe, stripe)])
        pltpu.sync_copy(wx_hbm.at[pl.ds(sid * stripe, stripe)],
                        wx_sp.at[pl.ds(sid * stripe, stripe)])
        plsc.subcore_barrier()

        def idx_load(b, t):
            return pltpu.make_async_copy(pos_hbm.at[b0 + b], idx_v.at[t],
                                         isems[t])

        def gathers_start(b_t, part, s):
            off, ln = _PARTS[part]
            pltpu.async_copy(wn_sp.at[idx_v.at[b_t, 0, pl.ds(off, ln)]],
                             bn_v.at[s, pl.ds(0, ln)], gsems[s])
            pltpu.async_copy(wx_sp.at[idx_v.at[b_t, 1, pl.ds(off, ln)]],
                             bx_v.at[s, pl.ds(0, ln)], gsems[s])

        def gathers_wait(part, s):
            _, ln = _PARTS[part]
            pltpu.make_async_copy(wn_hbm.at[pl.ds(0, ln)],
                                  bn_v.at[s, pl.ds(0, ln)], gsems[s]).wait()
            pltpu.make_async_copy(wx_hbm.at[pl.ds(0, ln)],
                                  bx_v.at[s, pl.ds(0, ln)], gsems[s]).wait()

        def store_now(b, part, s):
            off, ln = _PARTS[part]
            return pltpu.make_async_copy(
                bn_v.at[s, pl.ds(0, ln)],
                out_hbm.at[pl.ds((b0 + b) * L + off, ln), 0], osems_n[s])

        def store_next(b, part, s):
            off, ln = _PARTS[part]
            return pltpu.make_async_copy(
                bx_v.at[s, pl.ds(0, ln)],
                out_hbm.at[pl.ds((b0 + b) * L + off, ln), 1], osems_x[s])

        idx_load(0, 0).start()
        idx_load(1, 1).start()

        @pl.loop(0, total_q, step=2 * _NP)
        def _body(g):
            for u in range(2 * _NP):
                q = g + u
                part = u % _NP
                s = u % 2
                b = q // _NP
                b_t = (u // _NP) % 2

                if part == 0:
                    idx_load(0, b_t).wait()

                @pl.when(q >= 2)
                def _():
                    store_now(0, part, s).wait()
                    store_next(0, part, s).wait()

                gathers_start(b_t, part, s)

                part_m1 = (u - 1) % _NP
                s_m1 = 1 - s
                b_t_m1 = ((u - 1) % (2 * _NP)) // _NP

                @pl.when(q >= 1)
                def _():
                    b_m1 = jnp.maximum(q - 1, 0) // _NP
                    gathers_wait(part_m1, s_m1)
                    store_now(b_m1, part_m1, s_m1).start()
                    store_next(b_m1, part_m1, s_m1).start()

                if part_m1 == _NP - 1:
                    @pl.when((q >= 1) & (q - 1 + 2 * _NP < total_q))
                    def _():
                        idx_load(jnp.maximum(q - 1, 0) // _NP + 2, b_t_m1).start()

        q_last = total_q - 1
        gathers_wait(q_last % _NP, q_last % 2)
        store_now(b_per_w - 1, q_last % _NP, q_last % 2).start()
        store_next(b_per_w - 1, q_last % _NP, q_last % 2).start()
        for q in (total_q - 2, total_q - 1):
            store_now(0, q % _NP, q % 2).wait()
            store_next(0, q % _NP, q % 2).wait()

    return k(pos, W_now, W_next)


def kernel(pos, W_now, W_next):
    B, _, L = pos.shape
    out = _sc_embed(pos.astype(jnp.int32), W_now, W_next, B, L)
    return out.reshape(B, L, 2 * _HALF)
